# baseline (speedup 1.0000x reference)
import functools

import jax
import jax.numpy as jnp
from jax import lax
from jax.experimental import pallas as pl
from jax.experimental.pallas import tpu as pltpu
from jax.experimental.pallas import tpu_sc as plsc

_B = 16384
_D = 64
_LANES = 16
_CHUNK = 128


def _rsqrt(x):
    i = lax.bitcast_convert_type(x, jnp.int32)
    i = jnp.int32(0x5F3759DF) - (i >> 1)
    y = lax.bitcast_convert_type(i, jnp.float32)
    for _ in range(3):
        y = y * (1.5 - 0.5 * x * y * y)
    return y


def kernel(H, r_tensor, ht, queries, rel_table):
    del queries
    hidx = ht[:, 0].astype(jnp.int32).reshape(_B // _CHUNK, _CHUNK)
    tidx = ht[:, 1].astype(jnp.int32).reshape(_B // _CHUNK, _CHUNK)
    ridx = r_tensor.astype(jnp.int32).reshape(_B // _CHUNK, _CHUNK)

    info = plsc.get_sparse_core_info()
    nw = info.num_cores * info.num_subcores
    bpw = _B // nw
    cpw = bpw // _CHUNK
    mesh = plsc.VectorSubcoreMesh(core_axis_name="c", subcore_axis_name="s")

    @functools.partial(
        pl.kernel,
        out_type=jax.ShapeDtypeStruct((_B,), jnp.float32),
        mesh=mesh,
        compiler_params=pltpu.CompilerParams(
            needs_layout_passes=False, use_tc_tiling_on_sc=False),
        scratch_types=[
            pltpu.VMEM((cpw, _CHUNK), jnp.int32),
            pltpu.VMEM((cpw, _CHUNK), jnp.int32),
            pltpu.VMEM((cpw, _CHUNK), jnp.int32),
            pltpu.VMEM((bpw, _D), jnp.float32),
            pltpu.VMEM((bpw, _D), jnp.float32),
            pltpu.VMEM((bpw, _D), jnp.float32),
            pltpu.VMEM((bpw,), jnp.float32),
            pltpu.SemaphoreType.DMA,
        ],
    )
    def _k(h_hbm, hidx_hbm, tidx_hbm, ridx_hbm, rel_hbm, out_hbm,
           hidx_v, tidx_v, ridx_v, hrow_v, trow_v, rrow_v, dist_v, sem):
        wid = lax.axis_index("s") * info.num_cores + lax.axis_index("c")
        pltpu.sync_copy(hidx_hbm.at[pl.ds(wid * cpw, cpw)], hidx_v)
        pltpu.sync_copy(tidx_hbm.at[pl.ds(wid * cpw, cpw)], tidx_v)
        pltpu.sync_copy(ridx_hbm.at[pl.ds(wid * cpw, cpw)], ridx_v)
        copies = []
        for j in range(cpw):
            sl = pl.ds(j * _CHUNK, _CHUNK)
            copies.append(pltpu.async_copy(h_hbm.at[hidx_v.at[j]], hrow_v.at[sl], sem))
            copies.append(pltpu.async_copy(h_hbm.at[tidx_v.at[j]], trow_v.at[sl], sem))
            copies.append(pltpu.async_copy(rel_hbm.at[ridx_v.at[j]], rrow_v.at[sl], sem))
        for c in copies:
            c.wait()

        lane = lax.iota(jnp.int32, _LANES)

        def group(g, carry):
            rid = g * _LANES + lane
            z = jnp.zeros((_LANES,), jnp.float32)
            hh = tt = rr = hr = hxt = rxt = z
            for dcol in range(_D):
                dvec = jnp.full((_LANES,), dcol, jnp.int32)
                hv = plsc.load_gather(hrow_v, [rid, dvec])
                tv = plsc.load_gather(trow_v, [rid, dvec])
                rv = plsc.load_gather(rrow_v, [rid, dvec])
                hh = hh + hv * hv
                tt = tt + tv * tv
                rr = rr + rv * rv
                hr = hr + hv * rv
                hxt = hxt + hv * tv
                rxt = rxt + rv * tv
            ih = _rsqrt(jnp.maximum(hh, 1e-24))
            it = _rsqrt(jnp.maximum(tt, 1e-24))
            d2 = (hh * ih * ih + tt * it * it + rr
                  + 2.0 * (hr * ih - hxt * (ih * it) - rxt * it))
            d2 = jnp.maximum(d2, 0.0)
            plsc.store_scatter(dist_v, [rid], d2 * _rsqrt(jnp.maximum(d2, 1e-30)))
            return carry

        lax.fori_loop(0, bpw // _LANES, group, 0)
        pltpu.sync_copy(dist_v, out_hbm.at[pl.ds(wid * bpw, bpw)])

    return _k(H, hidx, tidx, ridx, rel_table)
